# baseline (speedup 1.0000x reference)
import functools

import jax
import jax.numpy as jnp
from jax import lax
from jax.experimental import pallas as pl
from jax.experimental.pallas import tpu as pltpu
from jax.experimental.pallas import tpu_sc as plsc

NUM_SEG = 16
N_ROWS = 320000
DIM = 128
LANES = 16
VPR = DIM // LANES

NC = 2
NS = 16
NW = NC * NS
CHUNK = N_ROWS // NW
BLK = 40
NBLK = CHUNK // BLK
NBUF = 16
BS_ITERS = 14


def _sc_partials(x, ids):
  mesh = plsc.VectorSubcoreMesh(core_axis_name="c", subcore_axis_name="s")

  @functools.partial(
      pl.kernel,
      mesh=mesh,
      out_type=jax.ShapeDtypeStruct((NW, NUM_SEG, DIM), jnp.float32),
      scratch_types=[
          pltpu.VMEM((CHUNK + LANES,), jnp.int32),
          pltpu.VMEM((NBUF, BLK, DIM), jnp.float32),
          pltpu.VMEM((NUM_SEG, DIM), jnp.float32),
          pltpu.VMEM((2 * LANES,), jnp.int32),
      ] + [pltpu.SemaphoreType.DMA] * (NBUF + 1),
  )
  def k(x_hbm, ids_hbm, out_hbm, ids_v, buf_v, acc_v, bounds_v, *sems):
    wid = lax.axis_index("s") * NC + lax.axis_index("c")
    base = wid * CHUNK

    def issue(b, q):
      pltpu.async_copy(x_hbm.at[pl.ds(base + b * BLK, BLK)],
                       buf_v.at[q], sems[q])

    def wait(b, q):
      pltpu.make_async_copy(x_hbm.at[pl.ds(base + b * BLK, BLK)],
                            buf_v.at[q], sems[q]).wait()

    for q in range(NBUF - 1):
      issue(q, q)
    ids_dst = ids_v.at[pl.ds(0, CHUNK)]
    ids_src = ids_hbm.at[pl.ds(base, CHUNK)]
    pltpu.async_copy(ids_src, ids_dst, sems[NBUF])

    neg = jnp.full((LANES,), -jnp.inf, jnp.float32)
    for s in range(NUM_SEG):
      for v in range(VPR):
        acc_v[s, pl.ds(v * LANES, LANES)] = neg

    pltpu.make_async_copy(ids_src, ids_dst, sems[NBUF]).wait()

    def id_at(i):
      return ids_v[pl.ds(i, LANES)][0]

    def lower_bound(s):
      def bs_body(_, c):
        lo, hi = c
        mid = lax.shift_right_logical(lo + hi, 1)
        lt = id_at(mid) < s
        return (jnp.where(lt, mid + 1, lo), jnp.where(lt, hi, mid))

      _, hi = lax.fori_loop(0, BS_ITERS, bs_body, (jnp.int32(0),
                                                   jnp.int32(CHUNK)))
      return hi

    s_vec = lax.iota(jnp.int32, LANES)
    zeros = jnp.zeros((LANES,), jnp.int32)
    bounds_vec = zeros
    for s in range(1, NUM_SEG):
      bounds_vec = jnp.where(s_vec == s, lower_bound(jnp.int32(s)),
                             bounds_vec)
    bounds_v[pl.ds(0, LANES)] = bounds_vec
    bounds_v[pl.ds(LANES, LANES)] = zeros + CHUNK

    def bv_at(i):
      return bounds_v[pl.ds(i, LANES)][0]

    def process(par, b):
      p0 = b * BLK
      sfirst = id_at(p0)
      fast = sfirst == id_at(p0 + BLK - 1)

      @pl.when(fast)
      def _():
        def row4(h, accs):
          r = 4 * h
          for j in range(4):
            accs = tuple(
                jnp.maximum(accs[v],
                            buf_v[par, r + j, pl.ds(v * LANES, LANES)])
                for v in range(VPR)
            )
          return accs

        init = tuple(
            acc_v[sfirst, pl.ds(v * LANES, LANES)] for v in range(VPR)
        )
        accs = lax.fori_loop(0, BLK // 4, row4, init)
        for v in range(VPR):
          acc_v[sfirst, pl.ds(v * LANES, LANES)] = accs[v]

      @pl.when(jnp.logical_not(fast))
      def _():
        def seg_body(s, carry):
          st = jnp.maximum(bv_at(s), p0)
          en = jnp.minimum(bv_at(s + 1), p0 + BLK)

          def row_body(r, accs):
            return tuple(
                jnp.maximum(accs[v],
                            buf_v[par, r, pl.ds(v * LANES, LANES)])
                for v in range(VPR)
            )

          init = tuple(
              acc_v[s, pl.ds(v * LANES, LANES)] for v in range(VPR)
          )
          accs = lax.fori_loop(st - p0, en - p0, row_body, init)
          for v in range(VPR):
            acc_v[s, pl.ds(v * LANES, LANES)] = accs[v]
          return carry

        lax.fori_loop(sfirst, id_at(p0 + BLK - 1) + 1, seg_body, 0)

    def step(t, carry):
      for q in range(NBUF):
        b = NBUF * t + q
        wait(b, q)
        nb = b + NBUF - 1
        nq = (q + NBUF - 1) % NBUF

        @pl.when(nb < NBLK)
        def _(nb=nb, nq=nq):
          issue(nb, nq)

        process(q, b)
      return carry

    lax.fori_loop(0, NBLK // NBUF, step, 0)

    for q in range(NBLK % NBUF):
      b = (NBLK // NBUF) * NBUF + q
      wait(b, q)
      process(q, b)

    pltpu.sync_copy(acc_v, out_hbm.at[wid])

  return k(x, ids)


def _combine(p_ref, o_ref):
  o_ref[...] = jnp.max(p_ref[...], axis=0)


def kernel(x, segment_ids):
  ids = segment_ids.astype(jnp.int32)
  partials = _sc_partials(x, ids)
  return pl.pallas_call(
      _combine,
      out_shape=jax.ShapeDtypeStruct((NUM_SEG, DIM), jnp.float32),
  )(partials)


# speedup vs baseline: 1.3304x; 1.3304x over previous
import functools

import jax
import jax.numpy as jnp
from jax import lax
from jax.experimental import pallas as pl
from jax.experimental.pallas import tpu as pltpu
from jax.experimental.pallas import tpu_sc as plsc

NUM_SEG = 16
N_ROWS = 320000
DIM = 128
LANES = 16
VPR = DIM // LANES

NC = 2
NS = 16
NW = NC * NS
CHUNK = N_ROWS // NW
BLK = 80
NBLK = CHUNK // BLK
NBUF = 8
BS_ITERS = 14


def _sc_partials(x, ids):
  mesh = plsc.VectorSubcoreMesh(core_axis_name="c", subcore_axis_name="s")

  @functools.partial(
      pl.kernel,
      mesh=mesh,
      out_type=jax.ShapeDtypeStruct((NW, NUM_SEG, DIM), jnp.float32),
      scratch_types=[
          pltpu.VMEM((CHUNK + LANES,), jnp.int32),
          pltpu.VMEM((NBUF, BLK, DIM), jnp.float32),
          pltpu.VMEM((NUM_SEG, DIM), jnp.float32),
          pltpu.VMEM((2 * LANES,), jnp.int32),
      ] + [pltpu.SemaphoreType.DMA] * (NBUF + 1),
  )
  def k(x_hbm, ids_hbm, out_hbm, ids_v, buf_v, acc_v, bounds_v, *sems):
    wid = lax.axis_index("s") * NC + lax.axis_index("c")
    base = wid * CHUNK

    def issue(b, q):
      pltpu.async_copy(x_hbm.at[pl.ds(base + b * BLK, BLK)],
                       buf_v.at[q], sems[q])

    def wait(b, q):
      pltpu.make_async_copy(x_hbm.at[pl.ds(base + b * BLK, BLK)],
                            buf_v.at[q], sems[q]).wait()

    for q in range(NBUF - 1):
      issue(q, q)
    ids_dst = ids_v.at[pl.ds(0, CHUNK)]
    ids_src = ids_hbm.at[pl.ds(base, CHUNK)]
    pltpu.async_copy(ids_src, ids_dst, sems[NBUF])

    neg = jnp.full((LANES,), -jnp.inf, jnp.float32)
    for s in range(NUM_SEG):
      for v in range(VPR):
        acc_v[s, pl.ds(v * LANES, LANES)] = neg

    pltpu.make_async_copy(ids_src, ids_dst, sems[NBUF]).wait()

    def id_at(i):
      return ids_v[pl.ds(i, LANES)][0]

    def lower_bound(s):
      def bs_body(_, c):
        lo, hi = c
        mid = lax.shift_right_logical(lo + hi, 1)
        lt = id_at(mid) < s
        return (jnp.where(lt, mid + 1, lo), jnp.where(lt, hi, mid))

      _, hi = lax.fori_loop(0, BS_ITERS, bs_body, (jnp.int32(0),
                                                   jnp.int32(CHUNK)))
      return hi

    s_vec = lax.iota(jnp.int32, LANES)
    zeros = jnp.zeros((LANES,), jnp.int32)
    bounds_vec = zeros
    for s in range(1, NUM_SEG):
      bounds_vec = jnp.where(s_vec == s, lower_bound(jnp.int32(s)),
                             bounds_vec)
    bounds_v[pl.ds(0, LANES)] = bounds_vec
    bounds_v[pl.ds(LANES, LANES)] = zeros + CHUNK

    def bv_at(i):
      return bounds_v[pl.ds(i, LANES)][0]

    def process(par, b):
      p0 = b * BLK
      sfirst = id_at(p0)
      fast = sfirst == id_at(p0 + BLK - 1)

      @pl.when(fast)
      def _():
        def row4(h, accs):
          r = 4 * h
          for j in range(4):
            accs = tuple(
                jnp.maximum(accs[v],
                            buf_v[par, r + j, pl.ds(v * LANES, LANES)])
                for v in range(VPR)
            )
          return accs

        init = tuple(
            acc_v[sfirst, pl.ds(v * LANES, LANES)] for v in range(VPR)
        )
        accs = lax.fori_loop(0, BLK // 4, row4, init)
        for v in range(VPR):
          acc_v[sfirst, pl.ds(v * LANES, LANES)] = accs[v]

      @pl.when(jnp.logical_not(fast))
      def _():
        def seg_body(s, carry):
          st = jnp.maximum(bv_at(s), p0)
          en = jnp.minimum(bv_at(s + 1), p0 + BLK)

          def row_body(r, accs):
            return tuple(
                jnp.maximum(accs[v],
                            buf_v[par, r, pl.ds(v * LANES, LANES)])
                for v in range(VPR)
            )

          init = tuple(
              acc_v[s, pl.ds(v * LANES, LANES)] for v in range(VPR)
          )
          accs = lax.fori_loop(st - p0, en - p0, row_body, init)
          for v in range(VPR):
            acc_v[s, pl.ds(v * LANES, LANES)] = accs[v]
          return carry

        lax.fori_loop(sfirst, id_at(p0 + BLK - 1) + 1, seg_body, 0)

    def step(t, carry):
      for q in range(NBUF):
        b = NBUF * t + q
        wait(b, q)
        nb = b + NBUF - 1
        nq = (q + NBUF - 1) % NBUF

        @pl.when(nb < NBLK)
        def _(nb=nb, nq=nq):
          issue(nb, nq)

        process(q, b)
      return carry

    lax.fori_loop(0, NBLK // NBUF, step, 0)

    for q in range(NBLK % NBUF):
      b = (NBLK // NBUF) * NBUF + q
      wait(b, q)
      process(q, b)

    pltpu.sync_copy(acc_v, out_hbm.at[wid])

  return k(x, ids)


def _combine(p_ref, o_ref):
  o_ref[...] = jnp.max(p_ref[...], axis=0)


def kernel(x, segment_ids):
  ids = segment_ids.astype(jnp.int32)
  partials = _sc_partials(x, ids)
  return pl.pallas_call(
      _combine,
      out_shape=jax.ShapeDtypeStruct((NUM_SEG, DIM), jnp.float32),
  )(partials)
